# speedup vs baseline: 5.0426x; 5.0426x over previous
import functools

import jax
import jax.numpy as jnp
from jax import lax
from jax.experimental import pallas as pl
from jax.experimental.pallas import tpu as pltpu

N = 100000
DIM_H = 512
NUM_GRAPHS = 256
ROWS = 1000
NB = N // ROWS


def _pool_ffn_body(batch_ref, h_ref, W1_ref, b1_ref, W2_ref, b2_ref,
                   out_ref, acc_ref):
    i = pl.program_id(0)

    @pl.when(i == 0)
    def _init():
        acc_ref[...] = jnp.zeros_like(acc_ref)

    ids = batch_ref[0, 0, :]
    seg = lax.broadcasted_iota(jnp.int32, (NUM_GRAPHS, ROWS), 0)
    onehot = (ids[None, :] == seg).astype(jnp.float32)
    acc_ref[...] += jnp.dot(onehot, h_ref[...],
                            preferred_element_type=jnp.float32)

    @pl.when(i == NB - 1)
    def _ffn():
        s = acc_ref[...]
        z = jnp.maximum(jnp.dot(s, W1_ref[...],
                                preferred_element_type=jnp.float32)
                        + b1_ref[...], 0.0)
        out_ref[...] = jnp.dot(z, W2_ref[...],
                               preferred_element_type=jnp.float32) + b2_ref[...]


def _broadcast_body(batch_ref, vn_ref, out_ref):
    ids = batch_ref[0, 0, :]
    seg = lax.broadcasted_iota(jnp.int32, (ROWS, NUM_GRAPHS), 1)
    onehot = (ids[:, None] == seg).astype(jnp.float32)
    out_ref[...] = jnp.dot(onehot, vn_ref[...],
                           preferred_element_type=jnp.float32)


@jax.jit
def kernel(h, batch, W1, b1, W2, b2):
    batch3 = batch.astype(jnp.int32).reshape(NB, 1, ROWS)

    h_vn = pl.pallas_call(
        _pool_ffn_body,
        grid=(NB,),
        in_specs=[
            pl.BlockSpec((1, 1, ROWS), lambda i: (i, 0, 0)),
            pl.BlockSpec((ROWS, DIM_H), lambda i: (i, 0)),
            pl.BlockSpec((DIM_H, 2 * DIM_H), lambda i: (0, 0)),
            pl.BlockSpec((2 * DIM_H,), lambda i: (0,)),
            pl.BlockSpec((2 * DIM_H, DIM_H), lambda i: (0, 0)),
            pl.BlockSpec((DIM_H,), lambda i: (0,)),
        ],
        out_specs=pl.BlockSpec((NUM_GRAPHS, DIM_H), lambda i: (0, 0)),
        out_shape=jax.ShapeDtypeStruct((NUM_GRAPHS, DIM_H), jnp.float32),
        scratch_shapes=[pltpu.VMEM((NUM_GRAPHS, DIM_H), jnp.float32)],
    )(batch3, h, W1, b1, W2, b2)

    out = pl.pallas_call(
        _broadcast_body,
        grid=(NB,),
        in_specs=[
            pl.BlockSpec((1, 1, ROWS), lambda i: (i, 0, 0)),
            pl.BlockSpec((NUM_GRAPHS, DIM_H), lambda i: (0, 0)),
        ],
        out_specs=pl.BlockSpec((ROWS, DIM_H), lambda i: (i, 0)),
        out_shape=jax.ShapeDtypeStruct((N, DIM_H), jnp.float32),
    )(batch3, h_vn)
    return out
